# baseline (speedup 1.0000x reference)
import functools

import jax
import jax.numpy as jnp
from jax import lax
from jax.experimental import pallas as pl
from jax.experimental.pallas import tpu as pltpu
from jax.experimental.pallas import tpu_sc as plsc

N, P, C, T = 16, 16384, 80, 50
CHP = 2048
NCH = P // CHP
PBF = 4096
KB = P // PBF
PC = P * C


def _neg_focal(x):
    u = jnp.exp(jnp.minimum(x, 60.0))
    a = 1.0 + u
    s = u * pl.reciprocal(a, approx=True)
    return 0.75 * jnp.log(a) * s * s


def _pos_focal(x):
    u = jnp.exp(jnp.minimum(-x, 60.0))
    a = 1.0 + u
    s = u / a
    return 0.25 * jnp.log(a) * s * s


def _balanced_l1(diff):
    alpha, gamma, beta = 0.5, 1.5, 0.11
    b = 19.085536923187668
    small = alpha / b * (b * diff + 1.0) * jnp.log(b * diff / beta + 1.0) - alpha * diff
    big = gamma * diff + gamma / b - alpha * beta
    return jnp.where(diff < beta, small, big)


def _match_kernel(loc_ref, priors_ref, tgt_ref, tgt_t_ref,
                  lsum_ref, npos_ref, meta_ref, idx_ref, bt_ref):
    n = pl.program_id(0)
    tref = tgt_ref[0]
    ax1 = tref[:, 0:1]
    ay1 = tref[:, 1:2]
    ax2 = tref[:, 2:3]
    ay2 = tref[:, 3:4]
    area_a = (ax2 - ax1) * (ay2 - ay1)
    iota_t = jax.lax.broadcasted_iota(jnp.int32, (T, CHP), 0).astype(jnp.float32)

    def pass1(i, carry):
        rmax, bpi = carry
        base = i * CHP
        cx = priors_ref[0:1, pl.ds(base, CHP)]
        cy = priors_ref[1:2, pl.ds(base, CHP)]
        w = priors_ref[2:3, pl.ds(base, CHP)]
        h = priors_ref[3:4, pl.ds(base, CHP)]
        bx1 = cx - 0.5 * w
        by1 = cy - 0.5 * h
        bx2 = cx + 0.5 * w
        by2 = cy + 0.5 * h
        iw = jnp.maximum(jnp.minimum(ax2, bx2) - jnp.maximum(ax1, bx1), 0.0)
        ih = jnp.maximum(jnp.minimum(ay2, by2) - jnp.maximum(ay1, by1), 0.0)
        inter = iw * ih
        ov = inter * pl.reciprocal(area_a + w * h - inter, approx=True)

        bto0 = jnp.max(ov, axis=0, keepdims=True)
        bti0 = jnp.min(jnp.where(ov == bto0, iota_t, float(T)),
                       axis=0, keepdims=True)
        bt_ref[0:1, pl.ds(base, CHP)] = bto0
        bt_ref[1:2, pl.ds(base, CHP)] = bti0

        iota_p = (jax.lax.broadcasted_iota(jnp.int32, (T, CHP), 1).astype(jnp.float32)
                  + base.astype(jnp.float32))
        cmax = jnp.max(ov, axis=1, keepdims=True)
        cbpi = jnp.min(jnp.where(ov == cmax, iota_p, float(P)),
                       axis=1, keepdims=True)
        take = cmax > rmax
        return jnp.maximum(rmax, cmax), jnp.where(take, cbpi, bpi)

    rmax0 = jnp.full((T, 1), -1.0, jnp.float32)
    bpi0 = jnp.zeros((T, 1), jnp.float32)
    _, bpi = jax.lax.fori_loop(0, NCH, pass1, (rmax0, bpi0))

    tref_t = tgt_t_ref[0]

    def pass2(i, carry):
        lsum, npos = carry
        base = i * CHP
        s_bto = bt_ref[0:1, pl.ds(base, CHP)]
        s_bti = bt_ref[1:2, pl.ds(base, CHP)]
        iota_p = (jax.lax.broadcasted_iota(jnp.int32, (T, CHP), 1).astype(jnp.float32)
                  + base.astype(jnp.float32))
        ft = jnp.max(jnp.where(bpi == iota_p, iota_t, -1.0),
                     axis=0, keepdims=True)
        forced = ft >= 0.0
        bto = jnp.where(forced, 2.0, s_bto)
        bti = jnp.where(forced, ft, s_bti)

        onehot = (iota_t == bti).astype(jnp.float32)
        g = jax.lax.dot_general(tref_t, onehot, (((1,), (0,)), ((), ())),
                                preferred_element_type=jnp.float32)
        gx1 = g[0:1, :]
        gy1 = g[1:2, :]
        gx2 = g[2:3, :]
        gy2 = g[3:4, :]
        glab = g[4:5, :]

        posf = (bto >= 0.5).astype(jnp.float32)
        ignf = ((bto >= 0.4) & (bto < 0.5)).astype(jnp.float32)
        neg = bto < 0.4
        conf_t = jnp.where(neg, 0.0, glab + 1.0)

        cx = priors_ref[0:1, pl.ds(base, CHP)]
        cy = priors_ref[1:2, pl.ds(base, CHP)]
        w = priors_ref[2:3, pl.ds(base, CHP)]
        h = priors_ref[3:4, pl.ds(base, CHP)]
        ecx = ((gx1 + gx2) * 0.5 - cx) / (0.1 * w)
        ecy = ((gy1 + gy2) * 0.5 - cy) / (0.1 * h)
        ew = jnp.log((gx2 - gx1) / w) * 5.0
        eh = jnp.log((gy2 - gy1) / h) * 5.0
        ll = (_balanced_l1(jnp.abs(loc_ref[0, 0:1, pl.ds(base, CHP)] - ecx))
              + _balanced_l1(jnp.abs(loc_ref[0, 1:2, pl.ds(base, CHP)] - ecy))
              + _balanced_l1(jnp.abs(loc_ref[0, 2:3, pl.ds(base, CHP)] - ew))
              + _balanced_l1(jnp.abs(loc_ref[0, 3:4, pl.ds(base, CHP)] - eh)))

        meta_ref[0, 0:1, pl.ds(base, CHP)] = posf
        meta_ref[0, 1:2, pl.ds(base, CHP)] = ignf
        meta_ref[0, 2:3, pl.ds(base, CHP)] = conf_t
        pglob = jax.lax.broadcasted_iota(jnp.int32, (1, CHP), 1) + base
        gidx = n * (C * P) + (conf_t.astype(jnp.int32) - 1) * P + pglob
        idx_ref[0, 0:1, pl.ds(base, CHP)] = jnp.maximum(gidx, 0)

        lsum = lsum + jnp.sum(ll * posf, keepdims=True)
        npos = npos + jnp.sum(posf, keepdims=True)
        return lsum, npos

    z = jnp.zeros((1, 1), jnp.float32)
    lsum, npos = jax.lax.fori_loop(0, NCH, pass2, (z, z))
    lsum_ref[0, :, :] = lsum
    npos_ref[0, :, :] = npos


def _matching_call(loc_t, priors_t, targets, targets_t, interpret=False):
    return pl.pallas_call(
        _match_kernel,
        grid=(N,),
        in_specs=[
            pl.BlockSpec((1, 4, P), lambda n: (n, 0, 0)),
            pl.BlockSpec((4, P), lambda n: (0, 0)),
            pl.BlockSpec((1, T, 5), lambda n: (n, 0, 0)),
            pl.BlockSpec((1, 5, T), lambda n: (n, 0, 0)),
        ],
        out_specs=[
            pl.BlockSpec((1, 1, 1), lambda n: (n, 0, 0)),
            pl.BlockSpec((1, 1, 1), lambda n: (n, 0, 0)),
            pl.BlockSpec((1, 3, P), lambda n: (n, 0, 0)),
            pl.BlockSpec((1, 1, P), lambda n: (n, 0, 0)),
        ],
        out_shape=[
            jax.ShapeDtypeStruct((N, 1, 1), jnp.float32),
            jax.ShapeDtypeStruct((N, 1, 1), jnp.float32),
            jax.ShapeDtypeStruct((N, 3, P), jnp.float32),
            jax.ShapeDtypeStruct((N, 1, P), jnp.int32),
        ],
        scratch_shapes=[pltpu.VMEM((8, P), jnp.float32)],
        interpret=interpret,
    )(loc_t, priors_t, targets, targets_t)


def _sc_gather_call(conf_flat, idx_flat):
    info = plsc.get_sparse_core_info()
    nc, ns = info.num_cores, info.num_subcores
    nw = nc * ns
    b_per_w = (N * P) // nw
    mesh = plsc.VectorSubcoreMesh(core_axis_name="c", subcore_axis_name="s")

    @functools.partial(
        pl.kernel, mesh=mesh,
        out_type=jax.ShapeDtypeStruct((N * P,), jnp.float32),
        scratch_types=[
            pltpu.VMEM((b_per_w,), jnp.int32),
            pltpu.VMEM((b_per_w,), jnp.float32),
            pltpu.SemaphoreType.DMA,
        ],
    )
    def _sc(table_hbm, idx_hbm, out_hbm, idx_v, rows_v, sem):
        wid = lax.axis_index("s") * nc + lax.axis_index("c")
        base = wid * b_per_w
        pltpu.sync_copy(idx_hbm.at[pl.ds(base, b_per_w)], idx_v)
        pltpu.async_copy(table_hbm.at[idx_v], rows_v, sem).wait()
        pltpu.sync_copy(rows_v, out_hbm.at[pl.ds(base, b_per_w)])

    return _sc(conf_flat, idx_flat)


def _focal_kernel(conf_ref, xs_ref, meta_ref, csum_ref):
    pb = pl.program_id(1)
    xt = conf_ref[0]
    csum = jnp.sum(_neg_focal(xt), keepdims=True)

    posf = meta_ref[0, 0:1, :]
    ignf = meta_ref[0, 1:2, :]
    xs = xs_ref[0, 0:1, :]
    spneg = _neg_focal(xs)
    corr = posf * (_pos_focal(xs) - spneg) - ignf * spneg
    csum = csum + jnp.sum(corr, keepdims=True)

    @pl.when(pb == 0)
    def _init():
        csum_ref[0, :, :] = jnp.zeros((1, 1), jnp.float32)

    csum_ref[0, :, :] += csum


def _focal_call(conf_t, xs_r, meta, interpret=False):
    return pl.pallas_call(
        _focal_kernel,
        grid=(N, KB),
        in_specs=[
            pl.BlockSpec((1, C, PBF), lambda n, pb: (n, 0, pb)),
            pl.BlockSpec((1, 1, PBF), lambda n, pb: (n, 0, pb)),
            pl.BlockSpec((1, 3, PBF), lambda n, pb: (n, 0, pb)),
        ],
        out_specs=pl.BlockSpec((1, 1, 1), lambda n, pb: (n, 0, 0)),
        out_shape=jax.ShapeDtypeStruct((N, 1, 1), jnp.float32),
        interpret=interpret,
    )(conf_t, xs_r, meta)


@jax.jit
def kernel(loc, conf, priors, targets):
    loc_t = jnp.transpose(loc, (0, 2, 1))
    conf_t = jnp.transpose(conf, (0, 2, 1))
    priors_t = jnp.transpose(priors, (1, 0))
    targets_t = jnp.transpose(targets, (0, 2, 1))
    lsum, npos, meta, idx = _matching_call(loc_t, priors_t, targets, targets_t)
    xs = _sc_gather_call(conf_t.reshape(-1), idx.reshape(-1))
    csum = _focal_call(conf_t, xs.reshape(N, 1, P), meta)
    num_pos = jnp.maximum(jnp.sum(npos), 1.0)
    return jnp.sum(lsum) / num_pos, jnp.sum(csum) / num_pos


# speedup vs baseline: 1.2680x; 1.2680x over previous
import jax
import jax.numpy as jnp
from jax.experimental import pallas as pl
from jax.experimental.pallas import tpu as pltpu

N, P, C, T = 16, 16384, 80, 50
CHP = 2048
NCH = P // CHP
PBF = 4096
KB = P // PBF


def _neg_focal(x):
    u = jnp.exp(jnp.minimum(x, 60.0))
    a = 1.0 + u
    s = u * pl.reciprocal(a, approx=True)
    return 0.75 * jnp.log(a) * s * s


def _pos_focal(x):
    u = jnp.exp(jnp.minimum(-x, 60.0))
    a = 1.0 + u
    s = u / a
    return 0.25 * jnp.log(a) * s * s


def _balanced_l1(diff):
    alpha, gamma, beta = 0.5, 1.5, 0.11
    b = 19.085536923187668
    small = alpha / b * (b * diff + 1.0) * jnp.log(b * diff / beta + 1.0) - alpha * diff
    big = gamma * diff + gamma / b - alpha * beta
    return jnp.where(diff < beta, small, big)


def _match_kernel(loc_ref, priors_ref, tgt_ref, tgt_t_ref,
                  lsum_ref, npos_ref, meta_ref, bt_ref):
    n = pl.program_id(0)
    tref = tgt_ref[0]
    ax1 = tref[:, 0:1]
    ay1 = tref[:, 1:2]
    ax2 = tref[:, 2:3]
    ay2 = tref[:, 3:4]
    area_a = (ax2 - ax1) * (ay2 - ay1)
    iota_t = jax.lax.broadcasted_iota(jnp.int32, (T, CHP), 0).astype(jnp.float32)

    def pass1(i, carry):
        rmax, bpi = carry
        base = i * CHP
        cx = priors_ref[0:1, pl.ds(base, CHP)]
        cy = priors_ref[1:2, pl.ds(base, CHP)]
        w = priors_ref[2:3, pl.ds(base, CHP)]
        h = priors_ref[3:4, pl.ds(base, CHP)]
        bx1 = cx - 0.5 * w
        by1 = cy - 0.5 * h
        bx2 = cx + 0.5 * w
        by2 = cy + 0.5 * h
        iw = jnp.maximum(jnp.minimum(ax2, bx2) - jnp.maximum(ax1, bx1), 0.0)
        ih = jnp.maximum(jnp.minimum(ay2, by2) - jnp.maximum(ay1, by1), 0.0)
        inter = iw * ih
        ov = inter * pl.reciprocal(area_a + w * h - inter, approx=True)

        bto0 = jnp.max(ov, axis=0, keepdims=True)
        bti0 = jnp.min(jnp.where(ov == bto0, iota_t, float(T)),
                       axis=0, keepdims=True)
        bt_ref[0:1, pl.ds(base, CHP)] = bto0
        bt_ref[1:2, pl.ds(base, CHP)] = bti0

        iota_p = (jax.lax.broadcasted_iota(jnp.int32, (T, CHP), 1).astype(jnp.float32)
                  + base.astype(jnp.float32))
        cmax = jnp.max(ov, axis=1, keepdims=True)
        cbpi = jnp.min(jnp.where(ov == cmax, iota_p, float(P)),
                       axis=1, keepdims=True)
        take = cmax > rmax
        return jnp.maximum(rmax, cmax), jnp.where(take, cbpi, bpi)

    rmax0 = jnp.full((T, 1), -1.0, jnp.float32)
    bpi0 = jnp.zeros((T, 1), jnp.float32)
    _, bpi = jax.lax.fori_loop(0, NCH, pass1, (rmax0, bpi0))

    tref_t = tgt_t_ref[0]

    def pass2(i, carry):
        lsum, npos = carry
        base = i * CHP
        s_bto = bt_ref[0:1, pl.ds(base, CHP)]
        s_bti = bt_ref[1:2, pl.ds(base, CHP)]
        iota_p = (jax.lax.broadcasted_iota(jnp.int32, (T, CHP), 1).astype(jnp.float32)
                  + base.astype(jnp.float32))
        ft = jnp.max(jnp.where(bpi == iota_p, iota_t, -1.0),
                     axis=0, keepdims=True)
        forced = ft >= 0.0
        bto = jnp.where(forced, 2.0, s_bto)
        bti = jnp.where(forced, ft, s_bti)

        onehot = (iota_t == bti).astype(jnp.float32)
        g = jax.lax.dot_general(tref_t, onehot, (((1,), (0,)), ((), ())),
                                preferred_element_type=jnp.float32)
        gx1 = g[0:1, :]
        gy1 = g[1:2, :]
        gx2 = g[2:3, :]
        gy2 = g[3:4, :]
        glab = g[4:5, :]

        posf = (bto >= 0.5).astype(jnp.float32)
        ignf = ((bto >= 0.4) & (bto < 0.5)).astype(jnp.float32)
        neg = bto < 0.4
        conf_t = jnp.where(neg, 0.0, glab + 1.0)

        cx = priors_ref[0:1, pl.ds(base, CHP)]
        cy = priors_ref[1:2, pl.ds(base, CHP)]
        w = priors_ref[2:3, pl.ds(base, CHP)]
        h = priors_ref[3:4, pl.ds(base, CHP)]
        ecx = ((gx1 + gx2) * 0.5 - cx) / (0.1 * w)
        ecy = ((gy1 + gy2) * 0.5 - cy) / (0.1 * h)
        ew = jnp.log((gx2 - gx1) / w) * 5.0
        eh = jnp.log((gy2 - gy1) / h) * 5.0
        ll = (_balanced_l1(jnp.abs(loc_ref[0, 0:1, pl.ds(base, CHP)] - ecx))
              + _balanced_l1(jnp.abs(loc_ref[0, 1:2, pl.ds(base, CHP)] - ecy))
              + _balanced_l1(jnp.abs(loc_ref[0, 2:3, pl.ds(base, CHP)] - ew))
              + _balanced_l1(jnp.abs(loc_ref[0, 3:4, pl.ds(base, CHP)] - eh)))

        meta_ref[0, 0:1, pl.ds(base, CHP)] = posf
        meta_ref[0, 1:2, pl.ds(base, CHP)] = ignf
        meta_ref[0, 2:3, pl.ds(base, CHP)] = conf_t

        lsum = lsum + jnp.sum(ll * posf, keepdims=True)
        npos = npos + jnp.sum(posf, keepdims=True)
        return lsum, npos

    z = jnp.zeros((1, 1), jnp.float32)
    lsum, npos = jax.lax.fori_loop(0, NCH, pass2, (z, z))
    lsum_ref[0, :, :] = lsum
    npos_ref[0, :, :] = npos


def _matching_call(loc_t, priors_t, targets, targets_t, interpret=False):
    return pl.pallas_call(
        _match_kernel,
        grid=(N,),
        in_specs=[
            pl.BlockSpec((1, 4, P), lambda n: (n, 0, 0)),
            pl.BlockSpec((4, P), lambda n: (0, 0)),
            pl.BlockSpec((1, T, 5), lambda n: (n, 0, 0)),
            pl.BlockSpec((1, 5, T), lambda n: (n, 0, 0)),
        ],
        out_specs=[
            pl.BlockSpec((1, 1, 1), lambda n: (n, 0, 0)),
            pl.BlockSpec((1, 1, 1), lambda n: (n, 0, 0)),
            pl.BlockSpec((1, 3, P), lambda n: (n, 0, 0)),
        ],
        out_shape=[
            jax.ShapeDtypeStruct((N, 1, 1), jnp.float32),
            jax.ShapeDtypeStruct((N, 1, 1), jnp.float32),
            jax.ShapeDtypeStruct((N, 3, P), jnp.float32),
        ],
        scratch_shapes=[pltpu.VMEM((8, P), jnp.float32)],
        interpret=interpret,
    )(loc_t, priors_t, targets, targets_t)


def _focal_kernel(conf_ref, meta_ref, csum_ref):
    n = pl.program_id(0)
    pb = pl.program_id(1)
    xt = conf_ref[0]
    negel = _neg_focal(xt)
    csum = jnp.sum(negel, keepdims=True)

    posf = meta_ref[0, 0:1, :]
    ignf = meta_ref[0, 1:2, :]
    ct = meta_ref[0, 2:3, :]
    iota_c = jax.lax.broadcasted_iota(jnp.int32, (C, PBF), 0).astype(jnp.float32)
    oh = (iota_c == ct - 1.0).astype(jnp.float32)
    xs = jnp.sum(oh * xt, axis=0, keepdims=True)
    spneg = _neg_focal(xs)
    corr = posf * (_pos_focal(xs) - spneg) - ignf * spneg
    csum = csum + jnp.sum(corr, keepdims=True)

    @pl.when(pb == 0)
    def _init():
        csum_ref[0, :, :] = jnp.zeros((1, 1), jnp.float32)

    csum_ref[0, :, :] += csum


def _focal_call(conf_t, meta, interpret=False):
    return pl.pallas_call(
        _focal_kernel,
        grid=(N, KB),
        in_specs=[
            pl.BlockSpec((1, C, PBF), lambda n, pb: (n, 0, pb)),
            pl.BlockSpec((1, 3, PBF), lambda n, pb: (n, 0, pb)),
        ],
        out_specs=pl.BlockSpec((1, 1, 1), lambda n, pb: (n, 0, 0)),
        out_shape=jax.ShapeDtypeStruct((N, 1, 1), jnp.float32),
        interpret=interpret,
    )(conf_t, meta)


@jax.jit
def kernel(loc, conf, priors, targets):
    loc_t = jnp.transpose(loc, (0, 2, 1))
    conf_t = jnp.transpose(conf, (0, 2, 1))
    priors_t = jnp.transpose(priors, (1, 0))
    targets_t = jnp.transpose(targets, (0, 2, 1))
    lsum, npos, meta = _matching_call(loc_t, priors_t, targets, targets_t)
    csum = _focal_call(conf_t, meta)
    num_pos = jnp.maximum(jnp.sum(npos), 1.0)
    return jnp.sum(lsum) / num_pos, jnp.sum(csum) / num_pos
